# baseline (speedup 1.0000x reference)
import functools

import jax
import jax.numpy as jnp
from jax import lax
from jax.experimental import pallas as pl
from jax.experimental.pallas import tpu as pltpu
from jax.experimental.pallas import tpu_sc as plsc

B = 16384
U = 1000000
D = 64

NC = 2
NS = 16
NW = NC * NS
BPW = B // NW
CHUNK = 128
NCHUNK = BPW // CHUNK

BLKU = 4096


def _tc_transpose_body(i0, i1, i2, i3, o0, o1, o2, o3):
    o0[...] = i0[...].T
    o1[...] = i1[...].T
    o2[...] = i2[...].T
    o3[...] = i3[...].T


def _tc_transpose(t0, t1, t2, t3):
    grid = (pl.cdiv(U, BLKU),)
    inspec = pl.BlockSpec((D, BLKU), lambda i: (0, i))
    outspec = pl.BlockSpec((BLKU, D), lambda i: (i, 0))
    return pl.pallas_call(
        _tc_transpose_body,
        grid=grid,
        in_specs=[inspec] * 4,
        out_specs=[outspec] * 4,
        out_shape=[jax.ShapeDtypeStruct((U, D), jnp.float32)] * 4,
    )(t0, t1, t2, t3)


def _sc_gather(ue_gmf, ie_gmf, ue_mlp, ie_mlp, user_r, item_r):
    mesh = plsc.VectorSubcoreMesh(core_axis_name="c", subcore_axis_name="s")
    out_t = tuple(jax.ShapeDtypeStruct((B, D), jnp.float32) for _ in range(4))

    @functools.partial(
        pl.kernel,
        out_type=out_t,
        mesh=mesh,
        compiler_params=pltpu.CompilerParams(use_tc_tiling_on_sc=False),
        scratch_types=[
            pltpu.VMEM((NCHUNK, CHUNK), jnp.int32),
            pltpu.VMEM((NCHUNK, CHUNK), jnp.int32),
            pltpu.VMEM((CHUNK, D), jnp.float32),
            pltpu.VMEM((CHUNK, D), jnp.float32),
            pltpu.SemaphoreType.DMA,
            pltpu.SemaphoreType.DMA,
        ],
    )
    def k(t0, t1, t2, t3, user_h, item_h,
          o_gu, o_gi, o_mu, o_mi, idx_u, idx_i, buf0, buf1, sem0, sem1):
        wid = lax.axis_index("s") * NC + lax.axis_index("c")
        base = wid * BPW
        pltpu.sync_copy(user_h.at[pl.ds(wid * NCHUNK, NCHUNK)], idx_u)
        pltpu.sync_copy(item_h.at[pl.ds(wid * NCHUNK, NCHUNK)], idx_i)

        ops = []
        for (tab, idx, out) in ((t0, idx_u, o_gu), (t1, idx_i, o_gi),
                                (t2, idx_u, o_mu), (t3, idx_i, o_mi)):
            for j in range(NCHUNK):
                ops.append((tab, idx, out, j))

        bufs = (buf0, buf1)
        sems = (sem0, sem1)
        n = len(ops)
        copies = [None] * n
        for kk in range(n):
            tab, idx, out, j = ops[kk]
            cp = pltpu.make_async_copy(
                tab.at[idx.at[j]], bufs[kk % 2], sems[kk % 2])
            cp.start()
            copies[kk] = cp
            if kk > 0:
                _, _, pout, pj = ops[kk - 1]
                copies[kk - 1].wait()
                pltpu.sync_copy(bufs[(kk - 1) % 2],
                                pout.at[pl.ds(base + pj * CHUNK, CHUNK)])
        _, _, out, j = ops[n - 1]
        copies[n - 1].wait()
        pltpu.sync_copy(bufs[(n - 1) % 2],
                        out.at[pl.ds(base + j * CHUNK, CHUNK)])

    return k(ue_gmf, ie_gmf, ue_mlp, ie_mlp, user_r, item_r)


BT = 1024


def _tc_mlp_body(gu_ref, gi_ref, mu_ref, mi_ref, w1a_ref, w1b_ref, b1_ref,
                 w2_ref, b2_ref, w3_ref, b3_ref, wog_ref, woh_ref, bo_ref,
                 out_ref):
    mu = mu_ref[...]
    mi = mi_ref[...]
    h1 = jnp.maximum(
        jnp.dot(mu, w1a_ref[...], preferred_element_type=jnp.float32)
        + jnp.dot(mi, w1b_ref[...], preferred_element_type=jnp.float32)
        + b1_ref[...], 0.0)
    h2 = jnp.maximum(
        jnp.dot(h1, w2_ref[...], preferred_element_type=jnp.float32)
        + b2_ref[...], 0.0)
    h3 = jnp.maximum(
        jnp.dot(h2, w3_ref[...], preferred_element_type=jnp.float32)
        + b3_ref[...], 0.0)
    gmf = gu_ref[...] * gi_ref[...]
    logit = (jnp.sum(gmf * wog_ref[...], axis=1)
             + jnp.sum(h3 * woh_ref[...], axis=1) + bo_ref[0, 0])
    out_ref[...] = jax.nn.sigmoid(logit)


def _tc_mlp(gu, gi, mu, mi, W1, b1, W2, b2, W3, b3, Wo, bo):
    w1t = W1.T
    w1a = w1t[:D]
    w1b = w1t[D:]
    w2t = W2.T
    w3t = jnp.pad(W3.T, ((0, 0), (0, 96)))
    b3p = jnp.pad(b3, (0, 96)).reshape(1, 128)
    wog = Wo[:, :D]
    woh = jnp.pad(Wo[:, D:], ((0, 0), (0, 96)))

    grid = (B // BT,)
    full = lambda shape: pl.BlockSpec(shape, lambda i: (0,) * len(shape))
    row = pl.BlockSpec((BT, D), lambda i: (i, 0))
    return pl.pallas_call(
        _tc_mlp_body,
        grid=grid,
        in_specs=[
            row, row, row, row,
            full((D, 128)), full((D, 128)), full((1, 128)),
            full((128, D)), full((1, D)),
            full((D, 128)), full((1, 128)),
            full((1, D)), full((1, 128)), full((1, 1)),
        ],
        out_specs=pl.BlockSpec((BT,), lambda i: (i,)),
        out_shape=jax.ShapeDtypeStruct((B,), jnp.float32),
    )(gu, gi, mu, mi, w1a, w1b, b1.reshape(1, 128), w2t,
      b2.reshape(1, D), w3t, b3p, wog, woh, bo.reshape(1, 1))


def kernel(user, item, ue_gmf, ie_gmf, ue_mlp, ie_mlp, W1, b1, W2, b2, W3, b3, Wo, bo):
    r0, r1, r2, r3 = _tc_transpose(ue_gmf.T, ie_gmf.T, ue_mlp.T, ie_mlp.T)
    user_r = user.astype(jnp.int32).reshape(NW * NCHUNK, CHUNK)
    item_r = item.astype(jnp.int32).reshape(NW * NCHUNK, CHUNK)
    gu, gi, mu, mi = _sc_gather(r0, r1, r2, r3, user_r, item_r)
    return _tc_mlp(gu, gi, mu, mi, W1, b1, W2, b2, W3, b3, Wo, bo)


# speedup vs baseline: 1.2075x; 1.2075x over previous
import functools

import jax
import jax.numpy as jnp
from jax import lax
from jax.experimental import pallas as pl
from jax.experimental.pallas import tpu as pltpu
from jax.experimental.pallas import tpu_sc as plsc

B = 16384
U = 1000000
D = 64

NC = 2
NS = 16
NW = NC * NS
BPW = B // NW
CHUNK = 128
NCHUNK = BPW // CHUNK

BLKU = 4096


def _tc_transpose_body(i0, i1, i2, i3, o0, o1, o2, o3):
    o0[...] = i0[...].T
    o1[...] = i1[...].T
    o2[...] = i2[...].T
    o3[...] = i3[...].T


def _tc_transpose(t0, t1, t2, t3):
    grid = (pl.cdiv(U, BLKU),)
    inspec = pl.BlockSpec((D, BLKU), lambda i: (0, i))
    outspec = pl.BlockSpec((BLKU, D), lambda i: (i, 0))
    return pl.pallas_call(
        _tc_transpose_body,
        grid=grid,
        in_specs=[inspec] * 4,
        out_specs=[outspec] * 4,
        out_shape=[jax.ShapeDtypeStruct((U, D), jnp.float32)] * 4,
    )(t0, t1, t2, t3)


def _sc_gather(ue_gmf, ie_gmf, ue_mlp, ie_mlp, user_r, item_r):
    mesh = plsc.VectorSubcoreMesh(core_axis_name="c", subcore_axis_name="s")
    out_t = tuple(jax.ShapeDtypeStruct((B, D), jnp.float32) for _ in range(4))

    @functools.partial(
        pl.kernel,
        out_type=out_t,
        mesh=mesh,
        compiler_params=pltpu.CompilerParams(use_tc_tiling_on_sc=False),
        scratch_types=[
            pltpu.VMEM((NCHUNK, CHUNK), jnp.int32),
            pltpu.VMEM((NCHUNK, CHUNK), jnp.int32),
            pltpu.VMEM((CHUNK, D), jnp.float32),
            pltpu.VMEM((CHUNK, D), jnp.float32),
            pltpu.SemaphoreType.DMA,
            pltpu.SemaphoreType.DMA,
        ],
    )
    def k(t0, t1, t2, t3, user_h, item_h,
          o_gu, o_gi, o_mu, o_mi, idx_u, idx_i, buf0, buf1, sem0, sem1):
        wid = lax.axis_index("s") * NC + lax.axis_index("c")
        base = wid * BPW
        pltpu.sync_copy(user_h.at[pl.ds(wid * NCHUNK, NCHUNK)], idx_u)
        pltpu.sync_copy(item_h.at[pl.ds(wid * NCHUNK, NCHUNK)], idx_i)

        ops = []
        for (tab, idx, out) in ((t0, idx_u, o_gu), (t1, idx_i, o_gi),
                                (t2, idx_u, o_mu), (t3, idx_i, o_mi)):
            for j in range(NCHUNK):
                ops.append((tab, idx, out, j))

        bufs = (buf0, buf1)
        sems = (sem0, sem1)
        n = len(ops)
        copies = [None] * n
        for kk in range(n):
            tab, idx, out, j = ops[kk]
            cp = pltpu.make_async_copy(
                tab.at[idx.at[j]], bufs[kk % 2], sems[kk % 2])
            cp.start()
            copies[kk] = cp
            if kk > 0:
                _, _, pout, pj = ops[kk - 1]
                copies[kk - 1].wait()
                pltpu.sync_copy(bufs[(kk - 1) % 2],
                                pout.at[pl.ds(base + pj * CHUNK, CHUNK)])
        _, _, out, j = ops[n - 1]
        copies[n - 1].wait()
        pltpu.sync_copy(bufs[(n - 1) % 2],
                        out.at[pl.ds(base + j * CHUNK, CHUNK)])

    return k(ue_gmf, ie_gmf, ue_mlp, ie_mlp, user_r, item_r)


BT = 1024


def _tc_mlp_body(gu_ref, gi_ref, mu_ref, mi_ref, w1a_ref, w1b_ref, b1_ref,
                 w2_ref, b2_ref, w3_ref, b3_ref, wog_ref, woh_ref, bo_ref,
                 out_ref):
    mu = mu_ref[...]
    mi = mi_ref[...]
    h1 = jnp.maximum(
        jnp.dot(mu, w1a_ref[...], preferred_element_type=jnp.float32)
        + jnp.dot(mi, w1b_ref[...], preferred_element_type=jnp.float32)
        + b1_ref[...], 0.0)
    h2 = jnp.maximum(
        jnp.dot(h1, w2_ref[...], preferred_element_type=jnp.float32)
        + b2_ref[...], 0.0)
    h3 = jnp.maximum(
        jnp.dot(h2, w3_ref[...], preferred_element_type=jnp.float32)
        + b3_ref[...], 0.0)
    gmf = gu_ref[...] * gi_ref[...]
    logit = (jnp.sum(gmf * wog_ref[...], axis=1)
             + jnp.sum(h3 * woh_ref[...], axis=1) + bo_ref[0, 0])
    out_ref[...] = jax.nn.sigmoid(logit)


def _tc_mlp(gu, gi, mu, mi, W1, b1, W2, b2, W3, b3, Wo, bo):
    w1t = W1.T
    w1a = w1t[:D]
    w1b = w1t[D:]
    w2t = W2.T
    w3t = jnp.pad(W3.T, ((0, 0), (0, 96)))
    b3p = jnp.pad(b3, (0, 96)).reshape(1, 128)
    wog = Wo[:, :D]
    woh = jnp.pad(Wo[:, D:], ((0, 0), (0, 96)))

    grid = (B // BT,)
    full = lambda shape: pl.BlockSpec(shape, lambda i: (0,) * len(shape))
    row = pl.BlockSpec((BT, D), lambda i: (i, 0))
    return pl.pallas_call(
        _tc_mlp_body,
        grid=grid,
        in_specs=[
            row, row, row, row,
            full((D, 128)), full((D, 128)), full((1, 128)),
            full((128, D)), full((1, D)),
            full((D, 128)), full((1, 128)),
            full((1, D)), full((1, 128)), full((1, 1)),
        ],
        out_specs=pl.BlockSpec((BT,), lambda i: (i,)),
        out_shape=jax.ShapeDtypeStruct((B,), jnp.float32),
    )(gu, gi, mu, mi, w1a, w1b, b1.reshape(1, 128), w2t,
      b2.reshape(1, D), w3t, b3p, wog, woh, bo.reshape(1, 1))


def _relax(x):
    y = jax.lax.optimization_barrier(x.reshape(B * D))
    return y.reshape(B, D)


def kernel(user, item, ue_gmf, ie_gmf, ue_mlp, ie_mlp, W1, b1, W2, b2, W3, b3, Wo, bo):
    user_r = user.astype(jnp.int32).reshape(NW * NCHUNK, CHUNK)
    item_r = item.astype(jnp.int32).reshape(NW * NCHUNK, CHUNK)
    gu, gi, mu, mi = (_relax(x) for x in _sc_gather(
        ue_gmf, ie_gmf, ue_mlp, ie_mlp, user_r, item_r))
    return _tc_mlp(gu, gi, mu, mi, W1, b1, W2, b2, W3, b3, Wo, bo)


# speedup vs baseline: 8.6090x; 7.1297x over previous
import functools

import jax
import jax.numpy as jnp
from jax import lax
from jax.experimental import pallas as pl
from jax.experimental.pallas import tpu as pltpu
from jax.experimental.pallas import tpu_sc as plsc

B = 16384
U = 1000000
D = 64

NC = 2
NS = 16
NW = NC * NS
BPW = B // NW
CHUNK = 128
NCHUNK = BPW // CHUNK

BLKU = 4096


def _tc_transpose_body(i0, i1, i2, i3, o0, o1, o2, o3):
    o0[...] = i0[...].T
    o1[...] = i1[...].T
    o2[...] = i2[...].T
    o3[...] = i3[...].T


def _tc_transpose(t0, t1, t2, t3):
    grid = (pl.cdiv(U, BLKU),)
    inspec = pl.BlockSpec((D, BLKU), lambda i: (0, i))
    outspec = pl.BlockSpec((BLKU, D), lambda i: (i, 0))
    return pl.pallas_call(
        _tc_transpose_body,
        grid=grid,
        in_specs=[inspec] * 4,
        out_specs=[outspec] * 4,
        out_shape=[jax.ShapeDtypeStruct((U, D), jnp.float32)] * 4,
    )(t0, t1, t2, t3)


def _sc_gather(t0_, t1_, t2_, t3_, user, item):
    mesh = plsc.VectorSubcoreMesh(core_axis_name="c", subcore_axis_name="s")
    out_t = tuple(jax.ShapeDtypeStruct((8, 8, B), jnp.float32)
                  for _ in range(4))

    @functools.partial(
        pl.kernel,
        out_type=out_t,
        mesh=mesh,
        scratch_types=[
            pltpu.VMEM((BPW,), jnp.int32),
            pltpu.VMEM((BPW,), jnp.int32),
            pltpu.VMEM((8, 8, 256), jnp.float32),
            pltpu.VMEM((8, 8, 256), jnp.float32),
            pltpu.VMEM((8, 8, 256), jnp.float32),
            pltpu.VMEM((8, 8, 256), jnp.float32),
            pltpu.SemaphoreType.DMA,
        ],
    )
    def k(t0, t1, t2, t3, user_h, item_h,
          o0, o1, o2, o3, idx_u, idx_i, b0, b1, b2, b3, sem):
        wid = lax.axis_index("s") * NC + lax.axis_index("c")
        base = wid * BPW
        pltpu.sync_copy(user_h.at[pl.ds(base, BPW)], idx_u)
        pltpu.sync_copy(item_h.at[pl.ds(base, BPW)], idx_i)

        def fetch16(idx, off, kk0, ta, bufa, tb, bufb):
            iv = idx[pl.ds(off, 16)]
            for l in range(16):
                u = pl.multiple_of(iv[l], 128)
                dsk = pl.ds(kk0 + l, 1)
                pltpu.make_async_copy(
                    ta.at[:, :, pl.ds(u, 1)], bufa.at[:, :, dsk],
                    sem).start()
                pltpu.make_async_copy(
                    tb.at[:, :, pl.ds(u, 1)], bufb.at[:, :, dsk],
                    sem).start()

        for j in range(2):
            def issue(i, _):
                off = j * 256 + i * 16
                fetch16(idx_u, off, i * 16, t0, b0, t2, b2)
                fetch16(idx_i, off, i * 16, t1, b1, t3, b3)
                return ()
            lax.fori_loop(0, 16, issue, (), unroll=False)
            for buf in (b0, b1, b2, b3):
                pltpu.make_async_copy(
                    t0.at[:, :, pl.ds(0, 256)], buf, sem).wait()
            for (buf, out) in ((b0, o0), (b1, o1), (b2, o2), (b3, o3)):
                pltpu.sync_copy(
                    buf, out.at[:, :, pl.ds(base + j * 256, 256)])

    return k(t0_, t1_, t2_, t3_, user, item)


BT = 1024


def _tc_mlp_body(gu_ref, gi_ref, mu_ref, mi_ref, w1a_ref, w1b_ref, b1_ref,
                 w2_ref, b2_ref, w3_ref, b3_ref, wog_ref, woh_ref, bo_ref,
                 out_ref):
    mu = mu_ref[...]
    mi = mi_ref[...]
    h1 = jnp.maximum(
        jnp.dot(mu, w1a_ref[...], preferred_element_type=jnp.float32)
        + jnp.dot(mi, w1b_ref[...], preferred_element_type=jnp.float32)
        + b1_ref[...], 0.0)
    h2 = jnp.maximum(
        jnp.dot(h1, w2_ref[...], preferred_element_type=jnp.float32)
        + b2_ref[...], 0.0)
    h3 = jnp.maximum(
        jnp.dot(h2, w3_ref[...], preferred_element_type=jnp.float32)
        + b3_ref[...], 0.0)
    gmf = gu_ref[...] * gi_ref[...]
    logit = (jnp.sum(gmf * wog_ref[...], axis=1)
             + jnp.sum(h3 * woh_ref[...], axis=1) + bo_ref[0, 0])
    out_ref[...] = jax.nn.sigmoid(logit)


def _tc_mlp(gu, gi, mu, mi, W1, b1, W2, b2, W3, b3, Wo, bo):
    w1t = W1.T
    w1a = w1t[:D]
    w1b = w1t[D:]
    w2t = W2.T
    w3t = jnp.pad(W3.T, ((0, 0), (0, 96)))
    b3p = jnp.pad(b3, (0, 96)).reshape(1, 128)
    wog = Wo[:, :D]
    woh = jnp.pad(Wo[:, D:], ((0, 0), (0, 96)))

    grid = (B // BT,)
    full = lambda shape: pl.BlockSpec(shape, lambda i: (0,) * len(shape))
    row = pl.BlockSpec((BT, D), lambda i: (i, 0))
    return pl.pallas_call(
        _tc_mlp_body,
        grid=grid,
        in_specs=[
            row, row, row, row,
            full((D, 128)), full((D, 128)), full((1, 128)),
            full((128, D)), full((1, D)),
            full((D, 128)), full((1, 128)),
            full((1, D)), full((1, 128)), full((1, 1)),
        ],
        out_specs=pl.BlockSpec((BT,), lambda i: (i,)),
        out_shape=jax.ShapeDtypeStruct((B,), jnp.float32),
    )(gu, gi, mu, mi, w1a, w1b, b1.reshape(1, 128), w2t,
      b2.reshape(1, D), w3t, b3p, wog, woh, bo.reshape(1, 1))


def _relax(x):
    y = jax.lax.optimization_barrier(x.reshape(B * D))
    return y.reshape(B, D)


def kernel(user, item, ue_gmf, ie_gmf, ue_mlp, ie_mlp, W1, b1, W2, b2, W3, b3, Wo, bo):
    tabs = tuple(t.T.reshape(8, 8, U)
                 for t in (ue_gmf, ie_gmf, ue_mlp, ie_mlp))
    outs = _sc_gather(*tabs, user.astype(jnp.int32), item.astype(jnp.int32))
    gu, gi, mu, mi = (x.reshape(D, B).T for x in outs)
    return _tc_mlp(gu, gi, mu, mi, W1, b1, W2, b2, W3, b3, Wo, bo)


# speedup vs baseline: 8.6612x; 1.0061x over previous
import functools

import jax
import jax.numpy as jnp
from jax import lax
from jax.experimental import pallas as pl
from jax.experimental.pallas import tpu as pltpu
from jax.experimental.pallas import tpu_sc as plsc

B = 16384
U = 1000000
D = 64

NC = 2
NS = 16
NW = NC * NS
BPW = B // NW
CHUNK = 128
NCHUNK = BPW // CHUNK

def _sc_gather(t0_, t1_, t2_, t3_, user, item):
    mesh = plsc.VectorSubcoreMesh(core_axis_name="c", subcore_axis_name="s")
    out_t = tuple(jax.ShapeDtypeStruct((8, 8, B), jnp.float32)
                  for _ in range(4))

    @functools.partial(
        pl.kernel,
        out_type=out_t,
        mesh=mesh,
        scratch_types=[
            pltpu.VMEM((BPW,), jnp.int32),
            pltpu.VMEM((BPW,), jnp.int32),
            pltpu.VMEM((8, 8, 256), jnp.float32),
            pltpu.VMEM((8, 8, 256), jnp.float32),
            pltpu.VMEM((8, 8, 256), jnp.float32),
            pltpu.VMEM((8, 8, 256), jnp.float32),
            pltpu.SemaphoreType.DMA,
        ],
    )
    def k(t0, t1, t2, t3, user_h, item_h,
          o0, o1, o2, o3, idx_u, idx_i, b0, b1, b2, b3, sem):
        wid = lax.axis_index("s") * NC + lax.axis_index("c")
        base = wid * BPW
        pltpu.sync_copy(user_h.at[pl.ds(base, BPW)], idx_u)
        pltpu.sync_copy(item_h.at[pl.ds(base, BPW)], idx_i)

        def fetch16(idx, off, kk0, ta, bufa, tb, bufb):
            iv = idx[pl.ds(off, 16)]
            for l in range(16):
                u = pl.multiple_of(iv[l], 128)
                dsk = pl.ds(kk0 + l, 1)
                pltpu.make_async_copy(
                    ta.at[:, :, pl.ds(u, 1)], bufa.at[:, :, dsk],
                    sem).start()
                pltpu.make_async_copy(
                    tb.at[:, :, pl.ds(u, 1)], bufb.at[:, :, dsk],
                    sem).start()

        for j in range(2):
            def issue(i, _):
                off = j * 256 + i * 16
                fetch16(idx_u, off, i * 16, t0, b0, t2, b2)
                fetch16(idx_i, off, i * 16, t1, b1, t3, b3)
                return ()
            lax.fori_loop(0, 16, issue, (), unroll=False)
            for buf in (b0, b1, b2, b3):
                pltpu.make_async_copy(
                    t0.at[:, :, pl.ds(0, 256)], buf, sem).wait()
            for (buf, out) in ((b0, o0), (b1, o1), (b2, o2), (b3, o3)):
                pltpu.sync_copy(
                    buf, out.at[:, :, pl.ds(base + j * 256, 256)])

    return k(t0_, t1_, t2_, t3_, user, item)


BT = 1024


def _tc_mlp_body(gu_ref, gi_ref, mu_ref, mi_ref, w1a_ref, w1b_ref, b1_ref,
                 w2_ref, b2_ref, w3_ref, b3_ref, wog_ref, woh_ref, bo_ref,
                 out_ref):
    mu = mu_ref[...]
    mi = mi_ref[...]
    h1 = jnp.maximum(
        jnp.dot(mu, w1a_ref[...], preferred_element_type=jnp.float32)
        + jnp.dot(mi, w1b_ref[...], preferred_element_type=jnp.float32)
        + b1_ref[...], 0.0)
    h2 = jnp.maximum(
        jnp.dot(h1, w2_ref[...], preferred_element_type=jnp.float32)
        + b2_ref[...], 0.0)
    h3 = jnp.maximum(
        jnp.dot(h2, w3_ref[...], preferred_element_type=jnp.float32)
        + b3_ref[...], 0.0)
    gmf = gu_ref[...] * gi_ref[...]
    logit = (jnp.sum(gmf * wog_ref[...], axis=1)
             + jnp.sum(h3 * woh_ref[...], axis=1) + bo_ref[0, 0])
    out_ref[...] = jax.nn.sigmoid(logit)


def _tc_mlp(gu, gi, mu, mi, W1, b1, W2, b2, W3, b3, Wo, bo):
    w1t = W1.T
    w1a = w1t[:D]
    w1b = w1t[D:]
    w2t = W2.T
    w3t = jnp.pad(W3.T, ((0, 0), (0, 96)))
    b3p = jnp.pad(b3, (0, 96)).reshape(1, 128)
    wog = Wo[:, :D]
    woh = jnp.pad(Wo[:, D:], ((0, 0), (0, 96)))

    grid = (B // BT,)
    full = lambda shape: pl.BlockSpec(shape, lambda i: (0,) * len(shape))
    row = pl.BlockSpec((BT, D), lambda i: (i, 0))
    return pl.pallas_call(
        _tc_mlp_body,
        grid=grid,
        in_specs=[
            row, row, row, row,
            full((D, 128)), full((D, 128)), full((1, 128)),
            full((128, D)), full((1, D)),
            full((D, 128)), full((1, 128)),
            full((1, D)), full((1, 128)), full((1, 1)),
        ],
        out_specs=pl.BlockSpec((BT,), lambda i: (i,)),
        out_shape=jax.ShapeDtypeStruct((B,), jnp.float32),
    )(gu, gi, mu, mi, w1a, w1b, b1.reshape(1, 128), w2t,
      b2.reshape(1, D), w3t, b3p, wog, woh, bo.reshape(1, 1))


def kernel(user, item, ue_gmf, ie_gmf, ue_mlp, ie_mlp, W1, b1, W2, b2, W3, b3, Wo, bo):
    tabs = tuple(t.T.reshape(8, 8, U)
                 for t in (ue_gmf, ie_gmf, ue_mlp, ie_mlp))
    outs = _sc_gather(*tabs, user.astype(jnp.int32), item.astype(jnp.int32))
    gu, gi, mu, mi = (x.reshape(D, B).T for x in outs)
    return _tc_mlp(gu, gi, mu, mi, W1, b1, W2, b2, W3, b3, Wo, bo)
